# baseline (speedup 1.0000x reference)
import functools

import jax
import jax.numpy as jnp
from jax import lax
from jax.experimental import pallas as pl
from jax.experimental.pallas import tpu as pltpu
from jax.experimental.pallas import tpu_sc as plsc

N = 10000
D = 128
E = 320000
G = 64
EPS = 1e-5

NC = 2
NS = 16
DH = D // NC
C = 128
K = 5
EPAD = 512
EP = 2560 * C
EDGES_PER_TILE = EP // NS
NBLK = EDGES_PER_TILE // (K * C)
IDX_ROWS_PER_TILE = EDGES_PER_TILE // C
ROW_STRIDE = 624
ROW_COPY = 640


def _sc_aggregate(h2, ei3, zeros):
    mesh = plsc.VectorSubcoreMesh(core_axis_name="c", subcore_axis_name="s")

    @functools.partial(
        pl.kernel,
        mesh=mesh,
        compiler_params=pltpu.CompilerParams(use_tc_tiling_on_sc=False),
        out_type=jax.ShapeDtypeStruct((NC, N, DH), jnp.float32),
        scratch_types=[
            pltpu.VMEM((2, K, C), jnp.int32),
            pltpu.VMEM((2, K, C), jnp.int32),
            pltpu.VMEM((2, K, C, DH), jnp.float32),
            pltpu.VMEM_SHARED((N + EPAD, DH), jnp.float32),
            pltpu.SemaphoreType.DMA,
            pltpu.SemaphoreType.DMA,
            pltpu.SemaphoreType.DMA,
        ],
    )
    def agg_kernel(h_hbm, ei_hbm, zeros_hbm, out_hbm,
                   src_v, dst_v, rows_v, acc_sh, sg0, sg1, ss):
        cid = lax.axis_index("c")
        sid = lax.axis_index("s")
        row0 = sid * ROW_STRIDE
        pltpu.sync_copy(zeros_hbm.at[pl.ds(row0, ROW_COPY)],
                        acc_sh.at[pl.ds(row0, ROW_COPY)])
        plsc.subcore_barrier()
        idx_base = sid * IDX_ROWS_PER_TILE
        sg = (sg0, sg1)
        hplane = h_hbm.at[cid]
        src_hbm = ei_hbm.at[0]
        dst_hbm = ei_hbm.at[1]

        def load_and_fire(b, p):
            roff = idx_base + b * K
            pltpu.sync_copy(src_hbm.at[pl.ds(roff, K)], src_v.at[p])
            pltpu.sync_copy(dst_hbm.at[pl.ds(roff, K)], dst_v.at[p])
            for j in range(K):
                pltpu.async_copy(hplane.at[src_v.at[p].at[j]],
                                 rows_v.at[p].at[j], sg[p])

        load_and_fire(0, 0)
        load_and_fire(1, 1)

        def body(i, carry):
            for p in range(2):
                b = 2 * i + p
                for j in range(K):
                    pltpu.make_async_copy(hplane.at[pl.ds(0, C)],
                                          rows_v.at[p].at[j], sg[p]).wait()
                scatters = [
                    pltpu.async_copy(rows_v.at[p].at[j],
                                     acc_sh.at[dst_v.at[p].at[j]],
                                     ss, add=True)
                    for j in range(K)
                ]
                for cp in scatters:
                    cp.wait()

                @pl.when(b + 2 < NBLK)
                def _():
                    load_and_fire(b + 2, p)
            return carry

        lax.fori_loop(0, NBLK // 2, body, 0)
        plsc.subcore_barrier()
        pltpu.sync_copy(acc_sh.at[pl.ds(row0, ROW_COPY)],
                        out_hbm.at[cid, pl.ds(row0, ROW_COPY)])

    return agg_kernel(h2, ei3, zeros)


def _combine_body(h_ref, a_ref, bn1_ref, b1n_ref,
                  Vw_ref, Vb_ref, Aw_ref, Ab_ref, Rw_ref, Rb_ref,
                  g_ref, be_ref, out_ref, *, final_refs=None):
    h = jnp.concatenate([h_ref[0], h_ref[1]], axis=1)
    aggr = jnp.concatenate([a_ref[0], a_ref[1]], axis=1)
    oh_ng = (bn1_ref[...] == lax.broadcasted_iota(jnp.int32, (N, G), 1)
             ).astype(jnp.float32)
    oh_gn = (b1n_ref[...] == lax.broadcasted_iota(jnp.int32, (G, N), 0)
             ).astype(jnp.float32)
    ro = jnp.dot(oh_gn, h, preferred_element_type=jnp.float32)
    roR = jnp.dot(ro, Rw_ref[...], preferred_element_type=jnp.float32)
    hpre = (jnp.dot(h, Vw_ref[...], preferred_element_type=jnp.float32)
            + jnp.dot(aggr, Aw_ref[...], preferred_element_type=jnp.float32)
            + jnp.dot(oh_ng, roR, preferred_element_type=jnp.float32)
            + Vb_ref[...] + Ab_ref[...] + Rb_ref[...])
    hr = jnp.maximum(hpre, 0.0)
    mu = jnp.mean(hr, axis=0, keepdims=True)
    var = jnp.mean((hr - mu) * (hr - mu), axis=0, keepdims=True)
    hbn = g_ref[...] * (hr - mu) * lax.rsqrt(var + EPS) + be_ref[...]
    if final_refs is None:
        out_ref[0] = hbn[:, :DH]
        out_ref[1] = hbn[:, DH:]
    else:
        Ww_ref, Wb_ref = final_refs
        out_ref[...] = (jnp.dot(hbn, Ww_ref[...],
                                preferred_element_type=jnp.float32)
                        + Wb_ref[...])


def _tc_combine(h, aggr, bn1, b1n, Vw, Vb, Aw, Ab, Rw, Rb, g, be,
                Ww=None, Wb=None):
    final = Ww is not None
    args = [h, aggr, bn1, b1n, Vw, Vb.reshape(1, D),
            Aw, Ab.reshape(1, D), Rw, Rb.reshape(1, D),
            g.reshape(1, D), be.reshape(1, D)]
    if final:
        args += [Ww, Wb.reshape(1, D)]

        def body(*refs):
            _combine_body(*refs[:12], refs[14], final_refs=(refs[12], refs[13]))
    else:
        def body(*refs):
            _combine_body(*refs, final_refs=None)

    out_shape = (jax.ShapeDtypeStruct((N, D), jnp.float32) if final
                 else jax.ShapeDtypeStruct((NC, N, DH), jnp.float32))
    return pl.pallas_call(
        body,
        out_shape=out_shape,
    )(*args)


def kernel(x, edge_index, batch, Vw0, Vb0, Aw0, Ab0, Rw0, Rb0, g0, be0,
           Vw1, Vb1, Aw1, Ab1, Rw1, Rb1, g1, be1, Ww, Wb):
    npad = EP - E
    pad_src = jnp.zeros((1, npad), dtype=jnp.int32)
    pad_dst = N + (jnp.arange(npad, dtype=jnp.int32) % EPAD).reshape(1, npad)
    pad = jnp.concatenate([pad_src, pad_dst], axis=0)
    ei3 = jnp.concatenate([edge_index, pad], axis=1).reshape(2, EP // C, C)
    zeros = jnp.zeros((N, DH), dtype=jnp.float32)
    bn1 = batch.reshape(N, 1)
    b1n = batch.reshape(1, N)
    x2 = x.reshape(N, NC, DH).transpose(1, 0, 2)

    a0 = _sc_aggregate(x2, ei3, zeros)
    h2 = _tc_combine(x2, a0, bn1, b1n, Vw0, Vb0, Aw0, Ab0, Rw0, Rb0, g0, be0)
    a1 = _sc_aggregate(h2, ei3, zeros)
    out = _tc_combine(h2, a1, bn1, b1n, Vw1, Vb1, Aw1, Ab1, Rw1, Rb1,
                      g1, be1, Ww, Wb)
    return out


# speedup vs baseline: 2.0827x; 2.0827x over previous
import functools

import jax
import jax.numpy as jnp
from jax import lax
from jax.experimental import pallas as pl
from jax.experimental.pallas import tpu as pltpu
from jax.experimental.pallas import tpu_sc as plsc

N = 10000
D = 128
E = 320000
G = 64
EPS = 1e-5

NC = 2
NS = 16
DH = D // NC
C = 128
K = 5
EPAD = 512
EP = 2560 * C
EDGES_PER_TILE = EP // NS
NBLK = EDGES_PER_TILE // (K * C)
IDX_ROWS_PER_TILE = EDGES_PER_TILE // C
ROW_STRIDE = 624
ROW_COPY = 640


def _sc_aggregate(h2, ei3, zeros):
    mesh = plsc.VectorSubcoreMesh(core_axis_name="c", subcore_axis_name="s")

    @functools.partial(
        pl.kernel,
        mesh=mesh,
        compiler_params=pltpu.CompilerParams(use_tc_tiling_on_sc=False),
        out_type=jax.ShapeDtypeStruct((NC, N, DH), jnp.float32),
        scratch_types=[
            pltpu.VMEM((2, K, C), jnp.int32),
            pltpu.VMEM((2, K, C), jnp.int32),
            pltpu.VMEM((2, K, C, DH), jnp.float32),
            pltpu.VMEM_SHARED((N + EPAD, DH), jnp.float32),
            pltpu.SemaphoreType.DMA,
            pltpu.SemaphoreType.DMA,
            pltpu.SemaphoreType.DMA,
        ],
    )
    def agg_kernel(h_hbm, ei_hbm, zeros_hbm, out_hbm,
                   src_v, dst_v, rows_v, acc_sh, sg0, sg1, ss):
        cid = lax.axis_index("c")
        sid = lax.axis_index("s")
        row0 = sid * ROW_STRIDE
        pltpu.sync_copy(zeros_hbm.at[pl.ds(row0, ROW_COPY)],
                        acc_sh.at[pl.ds(row0, ROW_COPY)])
        plsc.subcore_barrier()
        idx_base = sid * IDX_ROWS_PER_TILE
        sg = (sg0, sg1)
        hplane = h_hbm.at[cid]
        src_hbm = ei_hbm.at[0]
        dst_hbm = ei_hbm.at[1]

        def load_and_fire(b, p):
            roff = idx_base + b * K
            pltpu.sync_copy(src_hbm.at[pl.ds(roff, K)], src_v.at[p])
            pltpu.sync_copy(dst_hbm.at[pl.ds(roff, K)], dst_v.at[p])
            for j in range(K):
                pltpu.async_copy(hplane.at[src_v.at[p].at[j]],
                                 rows_v.at[p].at[j], sg[p])

        load_and_fire(0, 0)
        load_and_fire(1, 1)

        def body(i, carry):
            for p in range(2):
                b = 2 * i + p
                for j in range(K):
                    pltpu.make_async_copy(hplane.at[pl.ds(0, C)],
                                          rows_v.at[p].at[j], sg[p]).wait()
                scatters = [
                    pltpu.async_copy(rows_v.at[p].at[j],
                                     acc_sh.at[dst_v.at[p].at[j]],
                                     ss, add=True)
                    for j in range(K)
                ]
                for cp in scatters:
                    cp.wait()

                @pl.when(b + 2 < NBLK)
                def _():
                    load_and_fire(b + 2, p)
            return carry

        lax.fori_loop(0, NBLK // 2, body, 0)
        plsc.subcore_barrier()
        pltpu.sync_copy(acc_sh.at[pl.ds(row0, ROW_COPY)],
                        out_hbm.at[cid, pl.ds(row0, ROW_COPY)])

    return agg_kernel(h2, ei3, zeros)


def _combine_body(h_ref, a_ref, bn1_ref, b1n_ref,
                  Vw_ref, Vb_ref, Aw_ref, Ab_ref, Rw_ref, Rb_ref,
                  g_ref, be_ref, out_ref, *, final_refs=None):
    h = jnp.concatenate([h_ref[0], h_ref[1]], axis=1)
    aggr = jnp.concatenate([a_ref[0], a_ref[1]], axis=1)
    oh_ng = (bn1_ref[...] == lax.broadcasted_iota(jnp.int32, (N, G), 1)
             ).astype(jnp.float32)
    oh_gn = (b1n_ref[...] == lax.broadcasted_iota(jnp.int32, (G, N), 0)
             ).astype(jnp.float32)
    ro = jnp.dot(oh_gn, h, preferred_element_type=jnp.float32)
    roR = jnp.dot(ro, Rw_ref[...], preferred_element_type=jnp.float32)
    hpre = (jnp.dot(h, Vw_ref[...], preferred_element_type=jnp.float32)
            + jnp.dot(aggr, Aw_ref[...], preferred_element_type=jnp.float32)
            + jnp.dot(oh_ng, roR, preferred_element_type=jnp.float32)
            + Vb_ref[...] + Ab_ref[...] + Rb_ref[...])
    hr = jnp.maximum(hpre, 0.0)
    mu = jnp.mean(hr, axis=0, keepdims=True)
    var = jnp.mean((hr - mu) * (hr - mu), axis=0, keepdims=True)
    hbn = g_ref[...] * (hr - mu) * lax.rsqrt(var + EPS) + be_ref[...]
    if final_refs is None:
        out_ref[0] = hbn[:, :DH]
        out_ref[1] = hbn[:, DH:]
    else:
        Ww_ref, Wb_ref = final_refs
        out_ref[...] = (jnp.dot(hbn, Ww_ref[...],
                                preferred_element_type=jnp.float32)
                        + Wb_ref[...])


def _tc_combine(h, aggr, bn1, b1n, Vw, Vb, Aw, Ab, Rw, Rb, g, be,
                Ww=None, Wb=None):
    final = Ww is not None
    args = [h, aggr, bn1, b1n, Vw, Vb.reshape(1, D),
            Aw, Ab.reshape(1, D), Rw, Rb.reshape(1, D),
            g.reshape(1, D), be.reshape(1, D)]
    if final:
        args += [Ww, Wb.reshape(1, D)]

        def body(*refs):
            _combine_body(*refs[:12], refs[14], final_refs=(refs[12], refs[13]))
    else:
        def body(*refs):
            _combine_body(*refs, final_refs=None)

    out_shape = (jax.ShapeDtypeStruct((N, D), jnp.float32) if final
                 else jax.ShapeDtypeStruct((NC, N, DH), jnp.float32))
    return pl.pallas_call(
        body,
        out_shape=out_shape,
    )(*args)


def kernel(x, edge_index, batch, Vw0, Vb0, Aw0, Ab0, Rw0, Rb0, g0, be0,
           Vw1, Vb1, Aw1, Ab1, Rw1, Rb1, g1, be1, Ww, Wb):
    npad = EP - E
    pad_src = (jnp.arange(npad, dtype=jnp.int32) % N).reshape(1, npad)
    pad_dst = N + (jnp.arange(npad, dtype=jnp.int32) % EPAD).reshape(1, npad)
    pad = jnp.concatenate([pad_src, pad_dst], axis=0)
    ei3 = jnp.concatenate([edge_index, pad], axis=1).reshape(2, EP // C, C)
    zeros = jnp.zeros((N, DH), dtype=jnp.float32)
    bn1 = batch.reshape(N, 1)
    b1n = batch.reshape(1, N)
    x2 = x.reshape(N, NC, DH).transpose(1, 0, 2)

    a0 = _sc_aggregate(x2, ei3, zeros)
    h2 = _tc_combine(x2, a0, bn1, b1n, Vw0, Vb0, Aw0, Ab0, Rw0, Rb0, g0, be0)
    a1 = _sc_aggregate(h2, ei3, zeros)
    out = _tc_combine(h2, a1, bn1, b1n, Vw1, Vb1, Aw1, Ab1, Rw1, Rb1,
                      g1, be1, Ww, Wb)
    return out


# speedup vs baseline: 2.2249x; 1.0683x over previous
import functools

import jax
import jax.numpy as jnp
from jax import lax
from jax.experimental import pallas as pl
from jax.experimental.pallas import tpu as pltpu
from jax.experimental.pallas import tpu_sc as plsc

N = 10000
D = 128
E = 320000
G = 64
EPS = 1e-5

NC = 2
NS = 16
DH = D // NC
C = 128
K = 5
EPAD = 512
EP = 2560 * C
EDGES_PER_TILE = EP // NS
NBLK = EDGES_PER_TILE // (K * C)
IDX_ROWS_PER_TILE = EDGES_PER_TILE // C
ROW_STRIDE = 624
ROW_COPY = 640


def _sc_aggregate(h2, ei3, zeros):
    mesh = plsc.VectorSubcoreMesh(core_axis_name="c", subcore_axis_name="s")

    @functools.partial(
        pl.kernel,
        mesh=mesh,
        compiler_params=pltpu.CompilerParams(use_tc_tiling_on_sc=False),
        out_type=jax.ShapeDtypeStruct((N, D), jnp.float32),
        scratch_types=[
            pltpu.VMEM((2, K, C), jnp.int32),
            pltpu.VMEM((2, K, C), jnp.int32),
            pltpu.VMEM((2, K, C, DH), jnp.float32),
            pltpu.VMEM_SHARED((N + EPAD, DH), jnp.float32),
            pltpu.SemaphoreType.DMA,
            pltpu.SemaphoreType.DMA,
            pltpu.SemaphoreType.DMA,
        ],
    )
    def agg_kernel(h_hbm, ei_hbm, zeros_hbm, out_hbm,
                   src_v, dst_v, rows_v, acc_sh, sg0, sg1, ss):
        cid = lax.axis_index("c")
        sid = lax.axis_index("s")
        row0 = sid * ROW_STRIDE
        pltpu.sync_copy(zeros_hbm.at[pl.ds(row0, ROW_COPY)],
                        acc_sh.at[pl.ds(row0, ROW_COPY)])
        plsc.subcore_barrier()
        idx_base = sid * IDX_ROWS_PER_TILE
        sg = (sg0, sg1)
        hplane = h_hbm.at[cid]
        src_hbm = ei_hbm.at[0]
        dst_hbm = ei_hbm.at[1]

        def load_and_fire(b, p):
            roff = idx_base + b * K
            pltpu.sync_copy(src_hbm.at[pl.ds(roff, K)], src_v.at[p])
            pltpu.sync_copy(dst_hbm.at[pl.ds(roff, K)], dst_v.at[p])
            for j in range(K):
                pltpu.async_copy(hplane.at[src_v.at[p].at[j]],
                                 rows_v.at[p].at[j], sg[p])

        load_and_fire(0, 0)
        load_and_fire(1, 1)

        def body(i, carry):
            for p in range(2):
                b = 2 * i + p
                for j in range(K):
                    pltpu.make_async_copy(hplane.at[pl.ds(0, C)],
                                          rows_v.at[p].at[j], sg[p]).wait()
                scatters = [
                    pltpu.async_copy(rows_v.at[p].at[j],
                                     acc_sh.at[dst_v.at[p].at[j]],
                                     ss, add=True)
                    for j in range(K)
                ]
                for cp in scatters:
                    cp.wait()

                @pl.when(b + 2 < NBLK)
                def _():
                    load_and_fire(b + 2, p)
            return carry

        lax.fori_loop(0, NBLK // 2, body, 0)
        plsc.subcore_barrier()
        pltpu.sync_copy(acc_sh.at[pl.ds(row0, ROW_COPY)],
                        out_hbm.at[pl.ds(row0, ROW_COPY), pl.ds(cid * DH, DH)])

    return agg_kernel(h2, ei3, zeros)


def _combine_body(h_ref, a_ref, bn1_ref, b1n_ref,
                  Vw_ref, Vb_ref, Aw_ref, Ab_ref, Rw_ref, Rb_ref,
                  g_ref, be_ref, out_ref, *, final_refs=None):
    h = jnp.concatenate([h_ref[0], h_ref[1]], axis=1)
    aggr = a_ref[...]
    oh_ng = (bn1_ref[...] == lax.broadcasted_iota(jnp.int32, (N, G), 1)
             ).astype(jnp.float32)
    oh_gn = (b1n_ref[...] == lax.broadcasted_iota(jnp.int32, (G, N), 0)
             ).astype(jnp.float32)
    ro = jnp.dot(oh_gn, h, preferred_element_type=jnp.float32)
    roR = jnp.dot(ro, Rw_ref[...], preferred_element_type=jnp.float32)
    hpre = (jnp.dot(h, Vw_ref[...], preferred_element_type=jnp.float32)
            + jnp.dot(aggr, Aw_ref[...], preferred_element_type=jnp.float32)
            + jnp.dot(oh_ng, roR, preferred_element_type=jnp.float32)
            + Vb_ref[...] + Ab_ref[...] + Rb_ref[...])
    hr = jnp.maximum(hpre, 0.0)
    mu = jnp.mean(hr, axis=0, keepdims=True)
    var = jnp.mean((hr - mu) * (hr - mu), axis=0, keepdims=True)
    hbn = g_ref[...] * (hr - mu) * lax.rsqrt(var + EPS) + be_ref[...]
    if final_refs is None:
        out_ref[0] = hbn[:, :DH]
        out_ref[1] = hbn[:, DH:]
    else:
        Ww_ref, Wb_ref = final_refs
        out_ref[...] = (jnp.dot(hbn, Ww_ref[...],
                                preferred_element_type=jnp.float32)
                        + Wb_ref[...])


def _tc_combine(h, aggr, bn1, b1n, Vw, Vb, Aw, Ab, Rw, Rb, g, be,
                Ww=None, Wb=None):
    final = Ww is not None
    args = [h, aggr, bn1, b1n, Vw, Vb.reshape(1, D),
            Aw, Ab.reshape(1, D), Rw, Rb.reshape(1, D),
            g.reshape(1, D), be.reshape(1, D)]
    if final:
        args += [Ww, Wb.reshape(1, D)]

        def body(*refs):
            _combine_body(*refs[:12], refs[14], final_refs=(refs[12], refs[13]))
    else:
        def body(*refs):
            _combine_body(*refs, final_refs=None)

    out_shape = (jax.ShapeDtypeStruct((N, D), jnp.float32) if final
                 else jax.ShapeDtypeStruct((NC, N, DH), jnp.float32))
    return pl.pallas_call(
        body,
        out_shape=out_shape,
    )(*args)


def kernel(x, edge_index, batch, Vw0, Vb0, Aw0, Ab0, Rw0, Rb0, g0, be0,
           Vw1, Vb1, Aw1, Ab1, Rw1, Rb1, g1, be1, Ww, Wb):
    npad = EP - E
    pad_src = (jnp.arange(npad, dtype=jnp.int32) % N).reshape(1, npad)
    pad_dst = N + (jnp.arange(npad, dtype=jnp.int32) % EPAD).reshape(1, npad)
    pad = jnp.concatenate([pad_src, pad_dst], axis=0)
    ei3 = jnp.concatenate([edge_index, pad], axis=1).reshape(2, EP // C, C)
    zeros = jnp.zeros((N, DH), dtype=jnp.float32)
    bn1 = batch.reshape(N, 1)
    b1n = batch.reshape(1, N)
    x2 = x.reshape(N, NC, DH).transpose(1, 0, 2)

    a0 = _sc_aggregate(x2, ei3, zeros)
    h2 = _tc_combine(x2, a0, bn1, b1n, Vw0, Vb0, Aw0, Ab0, Rw0, Rb0, g0, be0)
    a1 = _sc_aggregate(h2, ei3, zeros)
    out = _tc_combine(h2, a1, bn1, b1n, Vw1, Vb1, Aw1, Ab1, Rw1, Rb1,
                      g1, be1, Ww, Wb)
    return out
